# baseline (speedup 1.0000x reference)
import functools

import jax
import jax.numpy as jnp
from jax import lax
from jax.experimental import pallas as pl
from jax.experimental.pallas import tpu as pltpu
from jax.experimental.pallas import tpu_sc as plsc

B, C, T, F, P, K, H = 32, 600, 250, 8, 8192, 20, 32
KF = K * F

_C_TILE = 75


def _featurize_body(sp_ref, wf_ref, bf_ref, out_ref):
    def one_cell(i, _):
        w = wf_ref[i]
        s = sp_ref[i]
        r = lax.dot_general(w, s, (((1,), (0,)), ((), ())),
                            preferred_element_type=jnp.float32)
        out_ref[i] = r + bf_ref[i]
        return 0

    lax.fori_loop(0, _C_TILE, one_cell, 0)


def _featurize(spikes_t, Wf, bf3):
    return pl.pallas_call(
        _featurize_body,
        grid=(C // _C_TILE,),
        in_specs=[
            pl.BlockSpec((_C_TILE, T, B), lambda i: (i, 0, 0)),
            pl.BlockSpec((_C_TILE, F, T), lambda i: (i, 0, 0)),
            pl.BlockSpec((_C_TILE, F, 1), lambda i: (i, 0, 0)),
        ],
        out_specs=pl.BlockSpec((_C_TILE, F, B), lambda i: (i, 0, 0)),
        out_shape=jax.ShapeDtypeStruct((C, F, B), jnp.float32),
    )(spikes_t, Wf, bf3)


_NC, _NS = 2, 16
_NW = _NC * _NS
_ROWS = P * K
_CH = 128
_NCH_TOTAL = _ROWS // _CH
_NCH = _NCH_TOTAL // _NW
_WROW = 128


def _sc_gather_body(table_ref, idx_ref, out_ref, idx_v, buf, sem):
    wid = lax.axis_index("s") * _NC + lax.axis_index("c")
    base_chunk = wid * _NCH
    pltpu.sync_copy(idx_ref.at[pl.ds(base_chunk, _NCH)], idx_v)

    def chunk(j, _):
        pltpu.async_copy(table_ref.at[idx_v.at[j]], buf, sem).wait()
        pltpu.sync_copy(buf, out_ref.at[pl.ds((base_chunk + j) * _CH, _CH)])
        return 0

    lax.fori_loop(0, _NCH, chunk, 0)


@functools.cache
def _sc_gather_kernel():
    return pl.kernel(
        _sc_gather_body,
        mesh=plsc.VectorSubcoreMesh(core_axis_name="c", subcore_axis_name="s",
                                    num_cores=_NC, num_subcores=_NS),
        out_type=jax.ShapeDtypeStruct((_ROWS, _WROW), jnp.int32),
        scratch_types=[
            pltpu.VMEM((_NCH, _CH), jnp.int32),
            pltpu.VMEM((_CH, _WROW), jnp.int32),
            pltpu.SemaphoreType.DMA,
        ],
    )


def _sc_gather(table, idx2):
    return _sc_gather_kernel()(table, idx2)


_PT = 128


def _mlp_body(x_ref, w1_ref, b1_ref, w2_ref, b2_ref, a_ref, out_ref):
    a = a_ref[0:1, 0:1]

    def pix(i, _):
        w1 = w1_ref[pl.ds(i * H, H), :].astype(jnp.bfloat16)
        x = x_ref[pl.ds(i * KF, KF), :]
        hid = lax.dot_general(w1, x, (((1,), (0,)), ((), ())),
                              preferred_element_type=jnp.float32)
        hid = hid + b1_ref[pl.ds(i * H, H), :]
        hid = jnp.maximum(hid, 0.0) + a * jnp.minimum(hid, 0.0)
        contrib = hid * w2_ref[pl.ds(i * H, H), :]
        row = jnp.sum(contrib, axis=0, keepdims=True) + b2_ref[pl.ds(i, 1), :]
        out_ref[pl.ds(i, 1), :] = row
        return 0

    lax.fori_loop(0, _PT, pix, 0)


def _mlp(xb, w1f, b1c, w2c, b2c, a2):
    return pl.pallas_call(
        _mlp_body,
        grid=(P // _PT,),
        in_specs=[
            pl.BlockSpec((_PT * KF, B), lambda i: (i, 0)),
            pl.BlockSpec((_PT * H, KF), lambda i: (i, 0)),
            pl.BlockSpec((_PT * H, 1), lambda i: (i, 0)),
            pl.BlockSpec((_PT * H, 1), lambda i: (i, 0)),
            pl.BlockSpec((_PT, 1), lambda i: (i, 0)),
            pl.BlockSpec((1, 1), lambda i: (0, 0)),
        ],
        out_specs=pl.BlockSpec((_PT, B), lambda i: (i, 0)),
        out_shape=jax.ShapeDtypeStruct((P, B), jnp.float32),
    )(xb, w1f, b1c, w2c, b2c, a2)


def kernel(time_binned_spikes, Wf, bf, pix_cell_sel, W1, b1, prelu_a, W2, b2):
    spikes_t = time_binned_spikes.transpose(1, 2, 0)
    bf3 = bf.reshape(C, F, 1)
    feat = _featurize(spikes_t, Wf, bf3)

    featb = feat.astype(jnp.bfloat16).reshape(C, _WROW, 2)
    table = lax.bitcast_convert_type(featb, jnp.int32)
    idx2 = pix_cell_sel.reshape(_NCH_TOTAL, _CH)
    xi = _sc_gather(table, idx2)

    xb = lax.bitcast_convert_type(xi, jnp.bfloat16)
    xb = xb.reshape(P * KF, B)

    w1f = W1.reshape(P * H, KF)
    b1c = b1.reshape(P * H, 1)
    w2c = W2.reshape(P * H, 1)
    b2c = b2.reshape(P, 1)
    a2 = prelu_a.reshape(1, 1)

    out_pb = _mlp(xb, w1f, b1c, w2c, b2c, a2)
    return out_pb.T


# speedup vs baseline: 1.0623x; 1.0623x over previous
import functools

import jax
import jax.numpy as jnp
from jax import lax
from jax.experimental import pallas as pl
from jax.experimental.pallas import tpu as pltpu
from jax.experimental.pallas import tpu_sc as plsc

B, C, T, F, P, K, H = 32, 600, 250, 8, 8192, 20, 32
KF = K * F

_C_TILE = 40


def _featurize_body(sp_ref, wf_ref, bf_ref, out_ref):
    r = lax.dot_general(wf_ref[...], sp_ref[...],
                        (((2,), (1,)), ((0,), (0,))),
                        preferred_element_type=jnp.float32)
    out_ref[...] = r + bf_ref[...]


def _featurize(spikes_t, Wf, bf3):
    return pl.pallas_call(
        _featurize_body,
        grid=(C // _C_TILE,),
        in_specs=[
            pl.BlockSpec((_C_TILE, T, B), lambda i: (i, 0, 0)),
            pl.BlockSpec((_C_TILE, F, T), lambda i: (i, 0, 0)),
            pl.BlockSpec((_C_TILE, F, 1), lambda i: (i, 0, 0)),
        ],
        out_specs=pl.BlockSpec((_C_TILE, F, B), lambda i: (i, 0, 0)),
        out_shape=jax.ShapeDtypeStruct((C, F, B), jnp.float32),
    )(spikes_t, Wf, bf3)


_NC, _NS = 2, 16
_NW = _NC * _NS
_ROWS = P * K
_CH = 128
_NCH_TOTAL = _ROWS // _CH
_NCH = _NCH_TOTAL // _NW
_WROW = 128


def _sc_gather_body(table_ref, idx_ref, out_ref, idx_v, buf0, buf1, sem0, sem1):
    wid = lax.axis_index("s") * _NC + lax.axis_index("c")
    base_chunk = wid * _NCH
    pltpu.sync_copy(idx_ref.at[pl.ds(base_chunk, _NCH)], idx_v)

    pltpu.make_async_copy(table_ref.at[idx_v.at[0]], buf0, sem0).start()

    def pair(i, _):
        j0 = i * 2
        pltpu.make_async_copy(table_ref.at[idx_v.at[j0 + 1]], buf1, sem1).start()
        pltpu.make_async_copy(table_ref.at[idx_v.at[j0]], buf0, sem0).wait()
        pltpu.sync_copy(buf0, out_ref.at[pl.ds((base_chunk + j0) * _CH, _CH)])

        @pl.when(j0 + 2 < _NCH)
        def _():
            pltpu.make_async_copy(table_ref.at[idx_v.at[j0 + 2]], buf0, sem0).start()

        pltpu.make_async_copy(table_ref.at[idx_v.at[j0 + 1]], buf1, sem1).wait()
        pltpu.sync_copy(buf1, out_ref.at[pl.ds((base_chunk + j0 + 1) * _CH, _CH)])
        return 0

    lax.fori_loop(0, _NCH // 2, pair, 0)


@functools.cache
def _sc_gather_kernel():
    return pl.kernel(
        _sc_gather_body,
        mesh=plsc.VectorSubcoreMesh(core_axis_name="c", subcore_axis_name="s",
                                    num_cores=_NC, num_subcores=_NS),
        out_type=jax.ShapeDtypeStruct((_ROWS, _WROW), jnp.int32),
        scratch_types=[
            pltpu.VMEM((_NCH, _CH), jnp.int32),
            pltpu.VMEM((_CH, _WROW), jnp.int32),
            pltpu.VMEM((_CH, _WROW), jnp.int32),
            pltpu.SemaphoreType.DMA,
            pltpu.SemaphoreType.DMA,
        ],
    )


def _sc_gather(table, idx2):
    return _sc_gather_kernel()(table, idx2)


_PT = 128


def _mlp_body(x_ref, w1_ref, b1_ref, w2_ref, b2_ref, a_ref, out_ref):
    a = a_ref[0:1, 0:1]
    x = x_ref[...].reshape(_PT, KF, B)
    w1 = w1_ref[...].astype(jnp.bfloat16).reshape(_PT, H, KF)
    hid = lax.dot_general(w1, x, (((2,), (1,)), ((0,), (0,))),
                          preferred_element_type=jnp.float32)
    hid = hid + b1_ref[...].reshape(_PT, H, 1)
    hid = jnp.maximum(hid, 0.0) + a[:, :, None] * jnp.minimum(hid, 0.0)
    contrib = hid * w2_ref[...].reshape(_PT, H, 1)
    res = jnp.sum(contrib, axis=1) + b2_ref[...]
    out_ref[...] = res.T


def _mlp(xb, w1f, b1c, w2c, b2c, a2):
    return pl.pallas_call(
        _mlp_body,
        grid=(P // _PT,),
        in_specs=[
            pl.BlockSpec((_PT * KF, B), lambda i: (i, 0)),
            pl.BlockSpec((_PT * H, KF), lambda i: (i, 0)),
            pl.BlockSpec((_PT * H, 1), lambda i: (i, 0)),
            pl.BlockSpec((_PT * H, 1), lambda i: (i, 0)),
            pl.BlockSpec((_PT, 1), lambda i: (i, 0)),
            pl.BlockSpec((1, 1), lambda i: (0, 0)),
        ],
        out_specs=pl.BlockSpec((B, _PT), lambda i: (0, i)),
        out_shape=jax.ShapeDtypeStruct((B, P), jnp.float32),
    )(xb, w1f, b1c, w2c, b2c, a2)


def kernel(time_binned_spikes, Wf, bf, pix_cell_sel, W1, b1, prelu_a, W2, b2):
    spikes_t = time_binned_spikes.transpose(1, 2, 0)
    bf3 = bf.reshape(C, F, 1)
    feat = _featurize(spikes_t, Wf, bf3)

    featb = feat.astype(jnp.bfloat16).reshape(C, _WROW, 2)
    table = lax.bitcast_convert_type(featb, jnp.int32)
    idx2 = pix_cell_sel.reshape(_NCH_TOTAL, _CH)
    xi = _sc_gather(table, idx2)

    xb = lax.bitcast_convert_type(xi, jnp.bfloat16)
    xb = xb.reshape(P * KF, B)

    w1f = W1.reshape(P * H, KF)
    b1c = b1.reshape(P * H, 1)
    w2c = W2.reshape(P * H, 1)
    b2c = b2.reshape(P, 1)
    a2 = prelu_a.reshape(1, 1)

    return _mlp(xb, w1f, b1c, w2c, b2c, a2)


# speedup vs baseline: 12.8224x; 12.0702x over previous
import functools

import jax
import jax.numpy as jnp
from jax import lax
from jax.experimental import pallas as pl
from jax.experimental.pallas import tpu as pltpu
from jax.experimental.pallas import tpu_sc as plsc

B, C, T, F, P, K, H = 32, 600, 250, 8, 8192, 20, 32
KF = K * F
FB = F * B

_C_TILE = 40


def _featurize_body(sp_ref, wf_ref, bf_ref, out_ref):
    r = lax.dot_general(wf_ref[...], sp_ref[...],
                        (((2,), (1,)), ((0,), (0,))),
                        preferred_element_type=jnp.float32)
    r = r + bf_ref[...]
    out_ref[...] = r.reshape(_C_TILE, FB)


def _featurize(spikes_t, Wf, bf3):
    return pl.pallas_call(
        _featurize_body,
        grid=(C // _C_TILE,),
        in_specs=[
            pl.BlockSpec((_C_TILE, T, B), lambda i: (i, 0, 0)),
            pl.BlockSpec((_C_TILE, F, T), lambda i: (i, 0, 0)),
            pl.BlockSpec((_C_TILE, F, 1), lambda i: (i, 0, 0)),
        ],
        out_specs=pl.BlockSpec((_C_TILE, FB), lambda i: (i, 0)),
        out_shape=jax.ShapeDtypeStruct((C, FB), jnp.float32),
    )(spikes_t, Wf, bf3)


_NC, _NS = 2, 16
_NW = _NC * _NS
_ROWS = P * K
_CH = 128
_NCH_TOTAL = _ROWS // _CH
_NCH = _NCH_TOTAL // _NW


def _sc_gather_body(table_ref, idx_ref, out_ref, idx_v, buf0, buf1, sem0, sem1):
    wid = lax.axis_index("s") * _NC + lax.axis_index("c")
    base_chunk = wid * _NCH
    pltpu.sync_copy(idx_ref.at[pl.ds(base_chunk, _NCH)], idx_v)

    pltpu.make_async_copy(table_ref.at[idx_v.at[0]], buf0, sem0).start()

    def pair(i, _):
        j0 = i * 2
        pltpu.make_async_copy(table_ref.at[idx_v.at[j0 + 1]], buf1, sem1).start()
        pltpu.make_async_copy(table_ref.at[idx_v.at[j0]], buf0, sem0).wait()
        pltpu.sync_copy(buf0, out_ref.at[pl.ds((base_chunk + j0) * _CH, _CH)])

        @pl.when(j0 + 2 < _NCH)
        def _():
            pltpu.make_async_copy(table_ref.at[idx_v.at[j0 + 2]], buf0, sem0).start()

        pltpu.make_async_copy(table_ref.at[idx_v.at[j0 + 1]], buf1, sem1).wait()
        pltpu.sync_copy(buf1, out_ref.at[pl.ds((base_chunk + j0 + 1) * _CH, _CH)])
        return 0

    lax.fori_loop(0, _NCH // 2, pair, 0)


@functools.cache
def _sc_gather_kernel():
    return pl.kernel(
        _sc_gather_body,
        mesh=plsc.VectorSubcoreMesh(core_axis_name="c", subcore_axis_name="s",
                                    num_cores=_NC, num_subcores=_NS),
        out_type=jax.ShapeDtypeStruct((_ROWS, FB), jnp.float32),
        scratch_types=[
            pltpu.VMEM((_NCH, _CH), jnp.int32),
            pltpu.VMEM((_CH, FB), jnp.float32),
            pltpu.VMEM((_CH, FB), jnp.float32),
            pltpu.SemaphoreType.DMA,
            pltpu.SemaphoreType.DMA,
        ],
    )


def _sc_gather(table, idx2):
    return _sc_gather_kernel()(table, idx2)


_PT = 128


def _mlp_body(x_ref, w1_ref, b1_ref, w2_ref, b2_ref, a_ref, out_ref):
    a = a_ref[0:1, 0:1]
    x3 = x_ref[...]
    parts = [x3[:, :, f * B:(f + 1) * B] for f in range(F)]
    xf = jnp.stack(parts, axis=2).reshape(_PT, KF, B)
    w1 = w1_ref[...]
    hid = lax.dot_general(w1, xf, (((2,), (1,)), ((0,), (0,))),
                          preferred_element_type=jnp.float32)
    hid = hid + b1_ref[...].reshape(_PT, H, 1)
    hid = jnp.maximum(hid, 0.0) + a[:, :, None] * jnp.minimum(hid, 0.0)
    contrib = hid * w2_ref[...].reshape(_PT, H, 1)
    res = jnp.sum(contrib, axis=1) + b2_ref[...].reshape(1, _PT).T
    out_ref[...] = res.T


def _mlp(xg, W1, b1, W2, b2r, a2):
    return pl.pallas_call(
        _mlp_body,
        grid=(P // _PT,),
        in_specs=[
            pl.BlockSpec((_PT, K, FB), lambda i: (i, 0, 0)),
            pl.BlockSpec((_PT, H, KF), lambda i: (i, 0, 0)),
            pl.BlockSpec((_PT, H), lambda i: (i, 0)),
            pl.BlockSpec((_PT, H), lambda i: (i, 0)),
            pl.BlockSpec((1, _PT), lambda i: (0, i)),
            pl.BlockSpec((1, 1), lambda i: (0, 0)),
        ],
        out_specs=pl.BlockSpec((B, _PT), lambda i: (0, i)),
        out_shape=jax.ShapeDtypeStruct((B, P), jnp.float32),
    )(xg, W1, b1, W2, b2r, a2)


def kernel(time_binned_spikes, Wf, bf, pix_cell_sel, W1, b1, prelu_a, W2, b2):
    spikes_t = time_binned_spikes.transpose(1, 2, 0)
    bf3 = bf.reshape(C, F, 1)
    table = _featurize(spikes_t, Wf, bf3)
    idx2 = pix_cell_sel.reshape(_NCH_TOTAL, _CH)
    xi = _sc_gather(table, idx2)

    xg = xi.reshape(P, K, FB)
    b2r = b2.reshape(1, P)
    a2 = prelu_a.reshape(1, 1)

    return _mlp(xg, W1, b1, W2, b2r, a2)
